# baseline (speedup 1.0000x reference)
import functools

import jax
import jax.numpy as jnp
from jax import lax
from jax.experimental import pallas as pl
from jax.experimental.pallas import tpu as pltpu
from jax.experimental.pallas import tpu_sc as plsc

NC = 2
NS = 16
NW = NC * NS
K = 80


def _make_deg(n, e):
  epw = e // NW
  nb = epw // K
  cpt = ((n // NS + 7) // 8) * 8
  mesh = plsc.VectorSubcoreMesh(core_axis_name="c", subcore_axis_name="s")

  @functools.partial(
      pl.kernel,
      mesh=mesh,
      out_type=jax.ShapeDtypeStruct((NC, n, 16), jnp.float32),
      scratch_types=[
          pltpu.VMEM((nb, K), jnp.int32),
          pltpu.VMEM((epw,), jnp.float32),
          pltpu.VMEM((K, 16), jnp.float32),
          pltpu.VMEM((K, 16), jnp.float32),
          pltpu.VMEM_SHARED((n, 16), jnp.float32),
          pltpu.SemaphoreType.DMA,
          pltpu.SemaphoreType.DMA,
      ],
      compiler_params=pltpu.CompilerParams(use_tc_tiling_on_sc=False),
  )
  def deg_kernel(dst_hbm, ew_hbm, out_hbm, dst_v, ew_v, msg0, msg1, acc,
                 ssem0, ssem1):
    c = lax.axis_index("c")
    s = lax.axis_index("s")
    wid = c * NS + s
    pltpu.sync_copy(dst_hbm.at[wid], dst_v)
    pltpu.sync_copy(ew_hbm.at[wid], ew_v)
    zero16 = jnp.zeros((16,), jnp.float32)
    for r in range(K):
      msg0[r, pl.ds(0, 16)] = zero16
      msg1[r, pl.ds(0, 16)] = zero16
    r0 = pl.multiple_of(jnp.minimum(s * cpt, n - cpt), 8)
    off = 0
    while off < cpt:
      nr = min(K, cpt - off)
      pltpu.sync_copy(msg0.at[pl.ds(0, nr)], acc.at[pl.ds(r0 + off, nr)])
      off += nr
    plsc.subcore_barrier()

    def build(msg, b):
      for jg in range(K // 16):
        w16 = ew_v[pl.ds(b * K + jg * 16, 16)]
        for i in range(16):
          msg[jg * 16 + i, pl.ds(0, 16)] = jnp.broadcast_to(w16[i], (16,))

    def sstart(msg, b, sem):
      pltpu.async_copy(msg, acc.at[dst_v.at[b]], sem, add=True)

    def swait(msg, sem):
      pltpu.make_async_copy(msg, acc.at[dst_v.at[0]], sem).wait()

    pltpu.async_copy(msg1, acc.at[dst_v.at[0]], ssem1, add=True)

    def body(i, carry):
      b0 = 2 * i
      b1 = b0 + 1
      swait(msg1, ssem1)
      build(msg0, b0)
      sstart(msg0, b0, ssem0)
      build(msg1, b1)
      swait(msg0, ssem0)
      sstart(msg1, b1, ssem1)
      return carry

    lax.fori_loop(0, (nb - 1) // 2, body, None)
    swait(msg1, ssem1)
    build(msg0, nb - 1)
    pltpu.sync_copy(msg0, acc.at[dst_v.at[nb - 1]], add=True)
    plsc.subcore_barrier()
    pltpu.sync_copy(acc.at[pl.ds(r0, cpt)], out_hbm.at[c, pl.ds(r0, cpt)])

  return deg_kernel


def _make_prop(n, e, d):
  epw = e // NW
  nb = epw // K
  cpt = ((n // NS + 7) // 8) * 8
  fgroups = d // 16
  mesh = plsc.VectorSubcoreMesh(core_axis_name="c", subcore_axis_name="s")

  @functools.partial(
      pl.kernel,
      mesh=mesh,
      out_type=jax.ShapeDtypeStruct((NC, n, d), jnp.float32),
      scratch_types=[
          pltpu.VMEM((epw,), jnp.int32),
          pltpu.VMEM((nb, K), jnp.int32),
          pltpu.VMEM((epw,), jnp.float32),
          pltpu.VMEM((K, d), jnp.float32),
          pltpu.VMEM((K, d), jnp.float32),
          pltpu.VMEM_SHARED((n, d), jnp.float32),
          pltpu.SemaphoreType.DMA,
          pltpu.SemaphoreType.DMA,
          pltpu.SemaphoreType.DMA,
          pltpu.SemaphoreType.DMA,
      ],
      compiler_params=pltpu.CompilerParams(use_tc_tiling_on_sc=False),
  )
  def prop(g_hbm, src_hbm, dst_hbm, ew_hbm, out_hbm,
           src_v, dst_v, ew_v, buf0, buf1, acc, gsem0, gsem1, ssem0, ssem1):
    c = lax.axis_index("c")
    s = lax.axis_index("s")
    wid = c * NS + s
    pltpu.sync_copy(src_hbm.at[wid], src_v)
    pltpu.sync_copy(dst_hbm.at[wid], dst_v)
    pltpu.sync_copy(ew_hbm.at[wid], ew_v)
    zero16 = jnp.zeros((16,), jnp.float32)
    for r in range(K):
      for f in range(fgroups):
        buf0[r, pl.ds(f * 16, 16)] = zero16
        buf1[r, pl.ds(f * 16, 16)] = zero16
    r0 = pl.multiple_of(jnp.minimum(s * cpt, n - cpt), 8)
    off = 0
    while off < cpt:
      nr = min(K, cpt - off)
      pltpu.sync_copy(buf0.at[pl.ds(0, nr)], acc.at[pl.ds(r0 + off, nr)])
      off += nr
    plsc.subcore_barrier()

    def gstart(b, buf, sem):
      pltpu.async_copy(g_hbm.at[src_v.at[pl.ds(b * K, K)]], buf, sem)

    def gwait(buf, sem):
      pltpu.make_async_copy(g_hbm.at[src_v.at[pl.ds(0, K)]], buf, sem).wait()

    def scale(buf, b):
      for jg in range(K // 16):
        w16 = ew_v[pl.ds(b * K + jg * 16, 16)]
        for i in range(16):
          j = jg * 16 + i
          wj = w16[i]
          for f in range(fgroups):
            sl = pl.ds(f * 16, 16)
            buf[j, sl] = buf[j, sl] * wj

    def sstart(buf, b, sem):
      pltpu.async_copy(buf, acc.at[dst_v.at[b]], sem, add=True)

    def swait(buf, sem):
      pltpu.make_async_copy(buf, acc.at[dst_v.at[0]], sem).wait()

    gstart(0, buf0, gsem0)
    pltpu.async_copy(buf1, acc.at[dst_v.at[0]], ssem1, add=True)

    def body(i, carry):
      b0 = 2 * i
      b1 = b0 + 1
      b2 = b0 + 2
      swait(buf1, ssem1)
      gstart(b1, buf1, gsem1)
      gwait(buf0, gsem0)
      scale(buf0, b0)
      sstart(buf0, b0, ssem0)
      gwait(buf1, gsem1)
      scale(buf1, b1)
      swait(buf0, ssem0)
      gstart(b2, buf0, gsem0)
      sstart(buf1, b1, ssem1)
      return carry

    lax.fori_loop(0, (nb - 1) // 2, body, None)
    swait(buf1, ssem1)
    gwait(buf0, gsem0)
    scale(buf0, nb - 1)
    pltpu.sync_copy(buf0, acc.at[dst_v.at[nb - 1]], add=True)
    plsc.subcore_barrier()
    pltpu.sync_copy(acc.at[pl.ds(r0, cpt)], out_hbm.at[c, pl.ds(r0, cpt)])

  return prop


_ROWS = 1000


def _dis_block(dp_ref):
  return lax.rsqrt(1.0 + dp_ref[0, :, 0:1] + dp_ref[1, :, 0:1])


def _b1(degparts, x, w1):
  n, d_in = x.shape
  d_h = w1.shape[1]

  def body(dp_ref, x_ref, w_ref, g_ref):
    dis = _dis_block(dp_ref)
    h = jnp.dot(x_ref[...], w_ref[...], preferred_element_type=jnp.float32)
    g_ref[...] = h * dis

  return pl.pallas_call(
      body,
      grid=(n // _ROWS,),
      in_specs=[
          pl.BlockSpec((NC, _ROWS, 16), lambda i: (0, i, 0)),
          pl.BlockSpec((_ROWS, d_in), lambda i: (i, 0)),
          pl.BlockSpec((d_in, d_h), lambda i: (0, 0)),
      ],
      out_specs=pl.BlockSpec((_ROWS, d_h), lambda i: (i, 0)),
      out_shape=jax.ShapeDtypeStruct((n, d_h), jnp.float32),
  )(degparts, x, w1)


def _b2(degparts, s1, g1, b1_2d, w2):
  n, d_h = g1.shape
  d_out = w2.shape[1]

  def body(dp_ref, s_ref, g_ref, b_ref, w_ref, o_ref):
    dis = _dis_block(dp_ref)
    t = s_ref[0] + s_ref[1] + g_ref[...]
    z = jnp.maximum(t * dis + b_ref[...], 0.0)
    o_ref[...] = jnp.dot(z, w_ref[...], preferred_element_type=jnp.float32) * dis

  return pl.pallas_call(
      body,
      grid=(n // _ROWS,),
      in_specs=[
          pl.BlockSpec((NC, _ROWS, 16), lambda i: (0, i, 0)),
          pl.BlockSpec((NC, _ROWS, d_h), lambda i: (0, i, 0)),
          pl.BlockSpec((_ROWS, d_h), lambda i: (i, 0)),
          pl.BlockSpec((1, d_h), lambda i: (0, 0)),
          pl.BlockSpec((d_h, d_out), lambda i: (0, 0)),
      ],
      out_specs=pl.BlockSpec((_ROWS, d_out), lambda i: (i, 0)),
      out_shape=jax.ShapeDtypeStruct((n, d_out), jnp.float32),
  )(degparts, s1, g1, b1_2d, w2)


def _b3(degparts, s2, g2, b2_2d):
  n, d_out = g2.shape

  def body(dp_ref, s_ref, g_ref, b_ref, o_ref):
    dis = _dis_block(dp_ref)
    o_ref[...] = (s_ref[0] + s_ref[1] + g_ref[...]) * dis + b_ref[...]

  return pl.pallas_call(
      body,
      grid=(n // _ROWS,),
      in_specs=[
          pl.BlockSpec((NC, _ROWS, 16), lambda i: (0, i, 0)),
          pl.BlockSpec((NC, _ROWS, d_out), lambda i: (0, i, 0)),
          pl.BlockSpec((_ROWS, d_out), lambda i: (i, 0)),
          pl.BlockSpec((1, d_out), lambda i: (0, 0)),
      ],
      out_specs=pl.BlockSpec((_ROWS, d_out), lambda i: (i, 0)),
      out_shape=jax.ShapeDtypeStruct((n, d_out), jnp.float32),
  )(degparts, s2, g2, b2_2d)


def kernel(x, edge_index, edge_weight, W1, b1, W2, b2):
  n = x.shape[0]
  e = edge_weight.shape[0]
  epw = e // NW
  nb = epw // K
  src = edge_index[0].reshape(NW, epw)
  dst = edge_index[1].reshape(NW, nb, K)
  ew = edge_weight.reshape(NW, epw)

  degparts = _make_deg(n, e)(dst, ew)
  g1 = _b1(degparts, x, W1)
  s1 = _make_prop(n, e, W1.shape[1])(g1, src, dst, ew)
  g2 = _b2(degparts, s1, g1, b1.reshape(1, -1), W2)
  s2 = _make_prop(n, e, W2.shape[1])(g2, src, dst, ew)
  out = _b3(degparts, s2, g2, b2.reshape(1, -1))
  return out


# speedup vs baseline: 1.0662x; 1.0662x over previous
import functools

import jax
import jax.numpy as jnp
from jax import lax
from jax.experimental import pallas as pl
from jax.experimental.pallas import tpu as pltpu
from jax.experimental.pallas import tpu_sc as plsc

NC = 2
NS = 16
NW = NC * NS
K = 80


def _make_deg(n, e):
  epw = e // NW
  nb = epw // K
  cpt = ((n // NS + 7) // 8) * 8
  mesh = plsc.VectorSubcoreMesh(core_axis_name="c", subcore_axis_name="s")

  @functools.partial(
      pl.kernel,
      mesh=mesh,
      out_type=jax.ShapeDtypeStruct((NC, n, 16), jnp.float32),
      scratch_types=[
          pltpu.VMEM((nb, K), jnp.int32),
          pltpu.VMEM((epw,), jnp.float32),
          pltpu.VMEM((K, 16), jnp.float32),
          pltpu.VMEM((K, 16), jnp.float32),
          pltpu.VMEM_SHARED((n, 16), jnp.float32),
          pltpu.SemaphoreType.DMA,
          pltpu.SemaphoreType.DMA,
      ],
      compiler_params=pltpu.CompilerParams(use_tc_tiling_on_sc=False),
  )
  def deg_kernel(dst_hbm, ew_hbm, out_hbm, dst_v, ew_v, msg0, msg1, acc,
                 ssem0, ssem1):
    c = lax.axis_index("c")
    s = lax.axis_index("s")
    wid = c * NS + s
    pltpu.sync_copy(dst_hbm.at[wid], dst_v)
    pltpu.sync_copy(ew_hbm.at[wid], ew_v)
    zero16 = jnp.zeros((16,), jnp.float32)
    for r in range(K):
      msg0[r, pl.ds(0, 16)] = zero16
      msg1[r, pl.ds(0, 16)] = zero16
    r0 = pl.multiple_of(jnp.minimum(s * cpt, n - cpt), 8)
    off = 0
    while off < cpt:
      nr = min(K, cpt - off)
      pltpu.sync_copy(msg0.at[pl.ds(0, nr)], acc.at[pl.ds(r0 + off, nr)])
      off += nr
    plsc.subcore_barrier()

    def build(msg, b):
      for jg in range(K // 16):
        w16 = ew_v[pl.ds(b * K + jg * 16, 16)]
        for i in range(16):
          msg[jg * 16 + i, pl.ds(0, 16)] = jnp.broadcast_to(w16[i], (16,))

    def sstart(msg, b, sem):
      pltpu.async_copy(msg, acc.at[dst_v.at[b]], sem, add=True)

    def swait(msg, sem):
      pltpu.make_async_copy(msg, acc.at[dst_v.at[0]], sem).wait()

    pltpu.async_copy(msg1, acc.at[dst_v.at[0]], ssem1, add=True)

    def body(i, carry):
      b0 = 2 * i
      b1 = b0 + 1
      swait(msg1, ssem1)
      build(msg0, b0)
      sstart(msg0, b0, ssem0)
      build(msg1, b1)
      swait(msg0, ssem0)
      sstart(msg1, b1, ssem1)
      return carry

    lax.fori_loop(0, (nb - 1) // 2, body, None)
    swait(msg1, ssem1)
    build(msg0, nb - 1)
    pltpu.sync_copy(msg0, acc.at[dst_v.at[nb - 1]], add=True)
    plsc.subcore_barrier()
    pltpu.sync_copy(acc.at[pl.ds(r0, cpt)], out_hbm.at[c, pl.ds(r0, cpt)])

  return deg_kernel


def _make_prop(n, e, d, nbuf):
  epw = e // NW
  nb = epw // K
  cpt = ((n // NS + 7) // 8) * 8
  fgroups = d // 16
  mesh = plsc.VectorSubcoreMesh(core_axis_name="c", subcore_axis_name="s")

  @functools.partial(
      pl.kernel,
      mesh=mesh,
      out_type=jax.ShapeDtypeStruct((NC, n, d), jnp.float32),
      scratch_types=[
          pltpu.VMEM((epw,), jnp.int32),
          pltpu.VMEM((nb, K), jnp.int32),
          pltpu.VMEM((epw,), jnp.float32),
          pltpu.VMEM_SHARED((n, d), jnp.float32),
      ] + [pltpu.VMEM((K, d), jnp.float32)] * nbuf
        + [pltpu.SemaphoreType.DMA] * (2 * nbuf),
      compiler_params=pltpu.CompilerParams(use_tc_tiling_on_sc=False),
  )
  def prop(g_hbm, src_hbm, dst_hbm, ew_hbm, out_hbm,
           src_v, dst_v, ew_v, acc, *bufsem):
    bufs = bufsem[:nbuf]
    gsems = bufsem[nbuf:2 * nbuf]
    ssems = bufsem[2 * nbuf:]
    c = lax.axis_index("c")
    s = lax.axis_index("s")
    wid = c * NS + s
    pltpu.sync_copy(src_hbm.at[wid], src_v)
    pltpu.sync_copy(dst_hbm.at[wid], dst_v)
    pltpu.sync_copy(ew_hbm.at[wid], ew_v)
    zero16 = jnp.zeros((16,), jnp.float32)
    for r in range(K):
      for f in range(fgroups):
        bufs[0][r, pl.ds(f * 16, 16)] = zero16
        bufs[-1][r, pl.ds(f * 16, 16)] = zero16
    r0 = pl.multiple_of(jnp.minimum(s * cpt, n - cpt), 8)
    off = 0
    while off < cpt:
      nr = min(K, cpt - off)
      pltpu.sync_copy(bufs[0].at[pl.ds(0, nr)], acc.at[pl.ds(r0 + off, nr)])
      off += nr
    plsc.subcore_barrier()

    def gstart(b, buf, sem):
      pltpu.async_copy(g_hbm.at[src_v.at[pl.ds(b * K, K)]], buf, sem)

    def gwait(buf, sem):
      pltpu.make_async_copy(g_hbm.at[src_v.at[pl.ds(0, K)]], buf, sem).wait()

    def scale(buf, b):
      for jg in range(K // 16):
        w16 = ew_v[pl.ds(b * K + jg * 16, 16)]
        for i in range(16):
          j = jg * 16 + i
          wj = w16[i]
          for f in range(fgroups):
            sl = pl.ds(f * 16, 16)
            buf[j, sl] = buf[j, sl] * wj

    def sstart(buf, b, sem):
      pltpu.async_copy(buf, acc.at[dst_v.at[b]], sem, add=True)

    def swait(buf, sem):
      pltpu.make_async_copy(buf, acc.at[dst_v.at[0]], sem).wait()

    for p in range(nbuf - 1):
      gstart(p, bufs[p], gsems[p])
    pltpu.async_copy(bufs[-1], acc.at[dst_v.at[0]], ssems[-1], add=True)

    if nbuf == 2:
      bufa, bufb = bufs
      gsa, gsb = gsems
      ssa, ssb = ssems

      def body(i, carry):
        b0 = 2 * i
        swait(bufb, ssb)
        gstart(b0 + 1, bufb, gsb)
        gwait(bufa, gsa)
        scale(bufa, b0)
        sstart(bufa, b0, ssa)
        gwait(bufb, gsb)
        scale(bufb, b0 + 1)
        swait(bufa, ssa)
        gstart(b0 + 2, bufa, gsa)
        sstart(bufb, b0 + 1, ssb)
        return carry

      lax.fori_loop(0, (nb - 1) // 2, body, None)
      swait(bufb, ssb)
      gwait(bufa, gsa)
      scale(bufa, nb - 1)
      pltpu.sync_copy(bufa, acc.at[dst_v.at[nb - 1]], add=True)
    else:
      bufa, bufb, bufc = bufs
      gsa, gsb, gsc = gsems
      ssa, ssb, ssc = ssems

      def body(i, carry):
        t0 = 3 * i
        swait(bufc, ssc)
        gstart(t0 + 2, bufc, gsc)
        gwait(bufa, gsa)
        scale(bufa, t0)
        sstart(bufa, t0, ssa)
        gwait(bufb, gsb)
        scale(bufb, t0 + 1)
        swait(bufa, ssa)
        gstart(t0 + 3, bufa, gsa)
        sstart(bufb, t0 + 1, ssb)
        gwait(bufc, gsc)
        scale(bufc, t0 + 2)
        swait(bufb, ssb)
        gstart(t0 + 4, bufb, gsb)
        sstart(bufc, t0 + 2, ssc)
        return carry

      lax.fori_loop(0, (nb - 2) // 3, body, None)
      swait(bufc, ssc)
      gwait(bufa, gsa)
      scale(bufa, nb - 2)
      sstart(bufa, nb - 2, ssa)
      gwait(bufb, gsb)
      scale(bufb, nb - 1)
      swait(bufa, ssa)
      pltpu.sync_copy(bufb, acc.at[dst_v.at[nb - 1]], add=True)

    plsc.subcore_barrier()
    pltpu.sync_copy(acc.at[pl.ds(r0, cpt)], out_hbm.at[c, pl.ds(r0, cpt)])

  return prop


_ROWS = 1000


def _dis_block(dp_ref):
  return lax.rsqrt(1.0 + dp_ref[0, :, 0:1] + dp_ref[1, :, 0:1])


def _b1(degparts, x, w1):
  n, d_in = x.shape
  d_h = w1.shape[1]

  def body(dp_ref, x_ref, w_ref, g_ref):
    dis = _dis_block(dp_ref)
    h = jnp.dot(x_ref[...], w_ref[...], preferred_element_type=jnp.float32)
    g_ref[...] = h * dis

  return pl.pallas_call(
      body,
      grid=(n // _ROWS,),
      in_specs=[
          pl.BlockSpec((NC, _ROWS, 16), lambda i: (0, i, 0)),
          pl.BlockSpec((_ROWS, d_in), lambda i: (i, 0)),
          pl.BlockSpec((d_in, d_h), lambda i: (0, 0)),
      ],
      out_specs=pl.BlockSpec((_ROWS, d_h), lambda i: (i, 0)),
      out_shape=jax.ShapeDtypeStruct((n, d_h), jnp.float32),
  )(degparts, x, w1)


def _b2(degparts, s1, g1, b1_2d, w2):
  n, d_h = g1.shape
  d_out = w2.shape[1]

  def body(dp_ref, s_ref, g_ref, b_ref, w_ref, o_ref):
    dis = _dis_block(dp_ref)
    t = s_ref[0] + s_ref[1] + g_ref[...]
    z = jnp.maximum(t * dis + b_ref[...], 0.0)
    o_ref[...] = jnp.dot(z, w_ref[...], preferred_element_type=jnp.float32) * dis

  return pl.pallas_call(
      body,
      grid=(n // _ROWS,),
      in_specs=[
          pl.BlockSpec((NC, _ROWS, 16), lambda i: (0, i, 0)),
          pl.BlockSpec((NC, _ROWS, d_h), lambda i: (0, i, 0)),
          pl.BlockSpec((_ROWS, d_h), lambda i: (i, 0)),
          pl.BlockSpec((1, d_h), lambda i: (0, 0)),
          pl.BlockSpec((d_h, d_out), lambda i: (0, 0)),
      ],
      out_specs=pl.BlockSpec((_ROWS, d_out), lambda i: (i, 0)),
      out_shape=jax.ShapeDtypeStruct((n, d_out), jnp.float32),
  )(degparts, s1, g1, b1_2d, w2)


def _b3(degparts, s2, g2, b2_2d):
  n, d_out = g2.shape

  def body(dp_ref, s_ref, g_ref, b_ref, o_ref):
    dis = _dis_block(dp_ref)
    o_ref[...] = (s_ref[0] + s_ref[1] + g_ref[...]) * dis + b_ref[...]

  return pl.pallas_call(
      body,
      grid=(n // _ROWS,),
      in_specs=[
          pl.BlockSpec((NC, _ROWS, 16), lambda i: (0, i, 0)),
          pl.BlockSpec((NC, _ROWS, d_out), lambda i: (0, i, 0)),
          pl.BlockSpec((_ROWS, d_out), lambda i: (i, 0)),
          pl.BlockSpec((1, d_out), lambda i: (0, 0)),
      ],
      out_specs=pl.BlockSpec((_ROWS, d_out), lambda i: (i, 0)),
      out_shape=jax.ShapeDtypeStruct((n, d_out), jnp.float32),
  )(degparts, s2, g2, b2_2d)


def kernel(x, edge_index, edge_weight, W1, b1, W2, b2):
  n = x.shape[0]
  e = edge_weight.shape[0]
  epw = e // NW
  nb = epw // K
  src = edge_index[0].reshape(NW, epw)
  dst = edge_index[1].reshape(NW, nb, K)
  ew = edge_weight.reshape(NW, epw)

  degparts = _make_deg(n, e)(dst, ew)
  g1 = _b1(degparts, x, W1)
  s1 = _make_prop(n, e, W1.shape[1], nbuf=2)(g1, src, dst, ew)
  g2 = _b2(degparts, s1, g1, b1.reshape(1, -1), W2)
  s2 = _make_prop(n, e, W2.shape[1], nbuf=3)(g2, src, dst, ew)
  out = _b3(degparts, s2, g2, b2.reshape(1, -1))
  return out
